# Initial kernel scaffold; baseline (speedup 1.0000x reference)
import jax
import jax.numpy as jnp
from jax.experimental import pallas as pl


def kernel(x, emb_weight, pe):
    raise NotImplementedError("write your pallas kernel here")


# speedup vs baseline: 1.2644x; 1.2644x over previous
import jax
import jax.numpy as jnp
from jax.experimental import pallas as pl
from jax.experimental.pallas import tpu as pltpu
from jax.experimental.pallas import tpu_sc as plsc

HIDDEN = 64
LANES = 16
WINDOW = 128


def _gather_scale(table, idx_flat):
    n = idx_flat.shape[0]
    idx2d = idx_flat.reshape(1, n)
    mesh = plsc.VectorSubcoreMesh(core_axis_name="c", subcore_axis_name="s")

    @pl.kernel(
        out_type=jax.ShapeDtypeStruct((n, HIDDEN), jnp.float32),
        mesh=mesh,
        compiler_params=pltpu.CompilerParams(use_tc_tiling_on_sc=False),
    )
    def k(table_hbm, idx_hbm, out_hbm):
        def body(i_vmem, o_vmem):
            pltpu.sync_copy(table_hbm.at[i_vmem.at[0]], o_vmem)

            @pl.loop(0, WINDOW)
            def _(r):
                for c in range(0, HIDDEN, LANES):
                    o_vmem[r, pl.ds(c, LANES)] = (
                        o_vmem[r, pl.ds(c, LANES)] * 8.0
                    )

        pltpu.emit_pipeline(
            body,
            grid=(n // WINDOW,),
            in_specs=[pl.BlockSpec((1, WINDOW), lambda i: (0, i))],
            out_specs=[pl.BlockSpec((WINDOW, HIDDEN), lambda i: (i, 0))],
            core_axis_name=("c", "s"),
            dimension_semantics=(pltpu.PARALLEL,),
        )(idx_hbm, out_hbm)

    return k(table, idx2d)


@jax.jit
def kernel(x, emb_weight, pe):
    del pe
    b, s = x.shape
    flat = _gather_scale(emb_weight, x.reshape(b * s).astype(jnp.int32))
    return flat.reshape(b, s, HIDDEN)


# speedup vs baseline: 1.8755x; 1.4833x over previous
import jax
import jax.numpy as jnp
from jax import lax
from jax.experimental import pallas as pl
from jax.experimental.pallas import tpu as pltpu
from jax.experimental.pallas import tpu_sc as plsc

HIDDEN = 64
LANES = 16
WINDOW = 128
NBUF = 4
NWORKERS = 32


def _gather_scale(table, idx_flat):
    n = idx_flat.shape[0]
    per_w = n // NWORKERS
    nchunk = per_w // WINDOW
    mesh = plsc.VectorSubcoreMesh(core_axis_name="c", subcore_axis_name="s")

    @pl.kernel(
        out_type=jax.ShapeDtypeStruct((n, HIDDEN), jnp.float32),
        mesh=mesh,
        compiler_params=pltpu.CompilerParams(use_tc_tiling_on_sc=False),
        scratch_types=(
            [pltpu.VMEM((per_w,), jnp.int32)]
            + [pltpu.VMEM((WINDOW, HIDDEN), jnp.float32)] * (2 * NBUF)
            + [pltpu.SemaphoreType.DMA] * (2 * NBUF)
        ),
    )
    def k(table_hbm, idx_hbm, out_hbm, idx_v, *rest):
        gbuf = rest[0:NBUF]
        obuf = rest[NBUF : 2 * NBUF]
        gsem = rest[2 * NBUF : 3 * NBUF]
        wsem = rest[3 * NBUF : 4 * NBUF]

        wid = lax.axis_index("s") * 2 + lax.axis_index("c")
        base = wid * per_w

        pltpu.sync_copy(idx_hbm.at[pl.ds(base, per_w)], idx_v)

        def gather_start(b, g):
            pltpu.make_async_copy(
                table_hbm.at[idx_v.at[pl.ds(g * WINDOW, WINDOW)]],
                gbuf[b],
                gsem[b],
            ).start()

        def gather_wait(b):
            pltpu.make_async_copy(
                table_hbm.at[idx_v.at[pl.ds(0, WINDOW)]], gbuf[b], gsem[b]
            ).wait()

        def write_start(b, g):
            pltpu.make_async_copy(
                obuf[b], out_hbm.at[pl.ds(base + g * WINDOW, WINDOW)], wsem[b]
            ).start()

        def write_wait(b):
            pltpu.make_async_copy(
                obuf[b], out_hbm.at[pl.ds(base, WINDOW)], wsem[b]
            ).wait()

        for b in range(NBUF):
            gather_start(b, b)

        @pl.loop(0, nchunk, step=NBUF)
        def _(g0):
            for b in range(NBUF):
                g = g0 + b
                gather_wait(b)

                @pl.when(g >= NBUF)
                def _(b=b):
                    write_wait(b)

                gb, ob = gbuf[b], obuf[b]

                @pl.loop(0, WINDOW)
                def _(r, gb=gb, ob=ob):
                    for c in range(0, HIDDEN, LANES):
                        ob[r, pl.ds(c, LANES)] = gb[r, pl.ds(c, LANES)] * 8.0

                write_start(b, g)

                @pl.when(g + NBUF < nchunk)
                def _(b=b, g=g):
                    gather_start(b, g + NBUF)

        for b in range(NBUF):
            write_wait(b)

    return k(table, idx_flat)


@jax.jit
def kernel(x, emb_weight, pe):
    del pe
    b, s = x.shape
    flat = _gather_scale(emb_weight, x.reshape(b * s).astype(jnp.int32))
    return flat.reshape(b, s, HIDDEN)


# speedup vs baseline: 1.8785x; 1.0016x over previous
import jax
import jax.numpy as jnp
from jax import lax
from jax.experimental import pallas as pl
from jax.experimental.pallas import tpu as pltpu
from jax.experimental.pallas import tpu_sc as plsc

HIDDEN = 64
LANES = 16
WINDOW = 128
NBUF = 4
NWORKERS = 32


def _gather_scale(table, idx_flat):
    n = idx_flat.shape[0]
    per_w = n // NWORKERS
    nchunk = per_w // WINDOW
    mesh = plsc.VectorSubcoreMesh(core_axis_name="c", subcore_axis_name="s")

    @pl.kernel(
        out_type=jax.ShapeDtypeStruct((n, HIDDEN), jnp.float32),
        mesh=mesh,
        compiler_params=pltpu.CompilerParams(use_tc_tiling_on_sc=False),
        scratch_types=(
            [pltpu.VMEM((per_w,), jnp.int32)]
            + [pltpu.VMEM((WINDOW, HIDDEN), jnp.float32)] * (2 * NBUF)
            + [pltpu.SemaphoreType.DMA] * (2 * NBUF)
        ),
    )
    def k(table_hbm, idx_hbm, out_hbm, idx_v, *rest):
        gbuf = rest[0:NBUF]
        obuf = rest[NBUF : 2 * NBUF]
        gsem = rest[2 * NBUF : 3 * NBUF]
        wsem = rest[3 * NBUF : 4 * NBUF]

        wid = lax.axis_index("s") * 2 + lax.axis_index("c")
        base = wid * per_w

        pltpu.sync_copy(idx_hbm.at[pl.ds(base, per_w)], idx_v)

        def gather_start(b, g):
            pltpu.make_async_copy(
                table_hbm.at[idx_v.at[pl.ds(g * WINDOW, WINDOW)]],
                gbuf[b],
                gsem[b],
            ).start()

        def gather_wait(b):
            pltpu.make_async_copy(
                table_hbm.at[idx_v.at[pl.ds(0, WINDOW)]], gbuf[b], gsem[b]
            ).wait()

        def write_start(b, g):
            pltpu.make_async_copy(
                obuf[b], out_hbm.at[pl.ds(base + g * WINDOW, WINDOW)], wsem[b]
            ).start()

        def write_wait(b):
            pltpu.make_async_copy(
                obuf[b], out_hbm.at[pl.ds(base, WINDOW)], wsem[b]
            ).wait()

        for b in range(NBUF):
            gather_start(b, b)

        @pl.loop(0, nchunk, step=NBUF)
        def _(g0):
            for b in range(NBUF):
                g = g0 + b
                gather_wait(b)

                @pl.when(g >= NBUF)
                def _(b=b):
                    write_wait(b)

                gb, ob = gbuf[b], obuf[b]

                @pl.loop(0, WINDOW, step=4)
                def _(r, gb=gb, ob=ob):
                    for rr in range(4):
                        for c in range(0, HIDDEN, LANES):
                            ob[r + rr, pl.ds(c, LANES)] = gb[r + rr, pl.ds(c, LANES)] * 8.0

                write_start(b, g)

                @pl.when(g + NBUF < nchunk)
                def _(b=b, g=g):
                    gather_start(b, g + NBUF)

        for b in range(NBUF):
            write_wait(b)

    return k(table, idx_flat)


@jax.jit
def kernel(x, emb_weight, pe):
    del pe
    b, s = x.shape
    flat = _gather_scale(emb_weight, x.reshape(b * s).astype(jnp.int32))
    return flat.reshape(b, s, HIDDEN)
